# baseline (speedup 1.0000x reference)
import jax
import jax.numpy as jnp
from jax import lax
from jax.experimental import pallas as pl
from jax.experimental.pallas import tpu as pltpu

B, N, M, D = 4, 4096, 4096, 3
K = 7
NC = 16
CR = N // NC
G = CR // 8


def _chamfer_body(xa_ref, ya_ref, out_ref):
    b = pl.program_id(0)
    scale = 1.0 / (B * N)

    @pl.when(b == 0)
    def _():
        out_ref[...] = jnp.zeros((1, 1), jnp.float32)

    ya = ya_ref[0]
    rowtotal = jnp.zeros((), jnp.float32)
    colmin8 = jnp.full((8, M), jnp.inf, jnp.float32)
    for c in range(NC):
        xc = xa_ref[0, c * CR:(c + 1) * CR, :]
        d2 = lax.dot_general(
            xc, ya, (((1,), (0,)), ((), ())),
            preferred_element_type=jnp.float32)

        rowmin = jnp.min(d2, axis=1)
        rowtotal += jnp.sum(jnp.maximum(rowmin, 0.0))
        colmin8 = jnp.minimum(
            colmin8, jnp.min(d2.reshape(G, 8, M), axis=0))

    cm = jnp.min(colmin8, axis=0)
    coltotal = jnp.sum(jnp.maximum(cm, 0.0))
    out_ref[...] += (rowtotal + coltotal) * scale


def _split_norm(n2):
    hi = n2.astype(jnp.bfloat16).astype(jnp.float32)
    return hi, n2 - hi


def kernel(x, y):
    x2h, x2l = _split_norm(jnp.sum(x * x, axis=-1, keepdims=True))
    y2h, y2l = _split_norm(jnp.sum(y * y, axis=-1, keepdims=True))
    ones = jnp.ones((B, N, 1), jnp.float32)
    xa = jnp.concatenate([-2.0 * x, ones, ones, x2h, x2l], -1)
    ya = jnp.transpose(
        jnp.concatenate([y, y2h, y2l, ones, ones], -1),
        (0, 2, 1))

    out = pl.pallas_call(
        _chamfer_body,
        grid=(B,),
        in_specs=[
            pl.BlockSpec((1, N, K), lambda b: (b, 0, 0)),
            pl.BlockSpec((1, K, M), lambda b: (b, 0, 0)),
        ],
        out_specs=pl.BlockSpec((1, 1), lambda b: (0, 0)),
        out_shape=jax.ShapeDtypeStruct((1, 1), jnp.float32),
        compiler_params=pltpu.CompilerParams(
            dimension_semantics=("arbitrary",)),
    )(xa, ya)
    return out[0, 0]


# speedup vs baseline: 1.3297x; 1.3297x over previous
import jax
import jax.numpy as jnp
from jax import lax
from jax.experimental import pallas as pl
from jax.experimental.pallas import tpu as pltpu

B, N, M, D = 4, 4096, 4096, 3
NC = 16
CR = N // NC
G = CR // 8


def _chamfer_body(x_ref, y_ref, out_ref):
    b = pl.program_id(0)

    ybt = jnp.swapaxes(y_ref[0], 0, 1)
    y2 = jnp.sum(ybt * ybt, axis=0)[None, :]
    scale = 1.0 / (B * N)

    @pl.when(b == 0)
    def _():
        out_ref[...] = jnp.zeros((1, 1), jnp.float32)

    rowtotal = jnp.zeros((), jnp.float32)
    colmin8 = jnp.full((8, M), jnp.inf, jnp.float32)
    for c in range(NC):
        xc = x_ref[0, c * CR:(c + 1) * CR, :]
        nxy2 = lax.dot_general(
            xc * -2.0, ybt, (((1,), (0,)), ((), ())),
            preferred_element_type=jnp.float32)
        x2 = jnp.sum(xc * xc, axis=1)[:, None]

        u = nxy2 + y2
        rowmin = jnp.min(u, axis=1)[:, None] + x2
        rowtotal += jnp.sum(jnp.maximum(rowmin, 0.0))

        w = nxy2 + x2
        colmin8 = jnp.minimum(
            colmin8, jnp.min(w.reshape(G, 8, M), axis=0))

    cm = jnp.min(colmin8, axis=0)[None, :] + y2
    coltotal = jnp.sum(jnp.maximum(cm, 0.0))
    out_ref[...] += (rowtotal + coltotal) * scale


def kernel(x, y):
    out = pl.pallas_call(
        _chamfer_body,
        grid=(B,),
        in_specs=[
            pl.BlockSpec((1, N, D), lambda b: (b, 0, 0)),
            pl.BlockSpec((1, M, D), lambda b: (b, 0, 0)),
        ],
        out_specs=pl.BlockSpec((1, 1), lambda b: (0, 0)),
        out_shape=jax.ShapeDtypeStruct((1, 1), jnp.float32),
        compiler_params=pltpu.CompilerParams(
            dimension_semantics=("arbitrary",)),
    )(x, y)
    return out.reshape(())


# speedup vs baseline: 1.4989x; 1.1272x over previous
import jax
import jax.numpy as jnp
from jax import lax
from jax.experimental import pallas as pl
from jax.experimental.pallas import tpu as pltpu

B, N, M, D = 4, 4096, 4096, 3
NC = 32
CR = N // NC
G = CR // 8


def _chamfer_body(x_ref, yt_ref, out_ref):
    b = pl.program_id(0)

    ybt = yt_ref[0]
    y2 = jnp.sum(ybt * ybt, axis=0)[None, :]
    scale = 1.0 / (B * N)

    @pl.when(b == 0)
    def _():
        out_ref[...] = jnp.zeros((1, 1), jnp.float32)

    rowtotal = jnp.zeros((), jnp.float32)
    colmin8 = jnp.full((8, M), jnp.inf, jnp.float32)
    for c in range(NC):
        xc = x_ref[0, c * CR:(c + 1) * CR, :]
        nxy2 = lax.dot_general(
            xc * -2.0, ybt, (((1,), (0,)), ((), ())),
            preferred_element_type=jnp.float32)
        x2 = jnp.sum(xc * xc, axis=1)[:, None]

        u = nxy2 + y2
        rowmin = jnp.min(u, axis=1)[:, None] + x2
        rowtotal += jnp.sum(jnp.maximum(rowmin, 0.0))

        w = nxy2 + x2
        colmin8 = jnp.minimum(
            colmin8, jnp.min(w.reshape(G, 8, M), axis=0))

    cm = jnp.min(colmin8, axis=0)[None, :] + y2
    coltotal = jnp.sum(jnp.maximum(cm, 0.0))
    out_ref[...] += (rowtotal + coltotal) * scale


def kernel(x, y):
    yt = jnp.transpose(y, (0, 2, 1))
    out = pl.pallas_call(
        _chamfer_body,
        grid=(B,),
        in_specs=[
            pl.BlockSpec((1, N, D), lambda b: (b, 0, 0)),
            pl.BlockSpec((1, D, M), lambda b: (b, 0, 0)),
        ],
        out_specs=pl.BlockSpec((1, 1), lambda b: (0, 0)),
        out_shape=jax.ShapeDtypeStruct((1, 1), jnp.float32),
        compiler_params=pltpu.CompilerParams(
            dimension_semantics=("arbitrary",)),
    )(x, yt)
    return out[0, 0]


# speedup vs baseline: 1.5178x; 1.0127x over previous
import jax
import jax.numpy as jnp
from jax import lax
from jax.experimental import pallas as pl
from jax.experimental.pallas import tpu as pltpu

B, N, M, D = 4, 4096, 4096, 3
NC = 64
CR = N // NC
G = CR // 8


def _chamfer_body(x_ref, yt_ref, out_ref):
    b = pl.program_id(0)

    ybt = yt_ref[0]
    y2 = jnp.sum(ybt * ybt, axis=0)[None, :]
    scale = 1.0 / (B * N)

    @pl.when(b == 0)
    def _():
        out_ref[...] = jnp.zeros((1, 1), jnp.float32)

    rowtotal = jnp.zeros((), jnp.float32)
    colmin8 = jnp.full((8, M), jnp.inf, jnp.float32)
    for c in range(NC):
        xc = x_ref[0, c * CR:(c + 1) * CR, :]
        nxy2 = lax.dot_general(
            xc * -2.0, ybt, (((1,), (0,)), ((), ())),
            preferred_element_type=jnp.float32)
        x2 = jnp.sum(xc * xc, axis=1)[:, None]

        u = nxy2 + y2
        rowmin = jnp.min(u, axis=1)[:, None] + x2
        rowtotal += jnp.sum(jnp.maximum(rowmin, 0.0))

        w = nxy2 + x2
        colmin8 = jnp.minimum(
            colmin8, jnp.min(w.reshape(G, 8, M), axis=0))

    cm = jnp.min(colmin8, axis=0)[None, :] + y2
    coltotal = jnp.sum(jnp.maximum(cm, 0.0))
    out_ref[...] += (rowtotal + coltotal) * scale


def kernel(x, y):
    yt = jnp.transpose(y, (0, 2, 1))
    out = pl.pallas_call(
        _chamfer_body,
        grid=(B,),
        in_specs=[
            pl.BlockSpec((1, N, D), lambda b: (b, 0, 0)),
            pl.BlockSpec((1, D, M), lambda b: (b, 0, 0)),
        ],
        out_specs=pl.BlockSpec((1, 1), lambda b: (0, 0)),
        out_shape=jax.ShapeDtypeStruct((1, 1), jnp.float32),
        compiler_params=pltpu.CompilerParams(
            dimension_semantics=("arbitrary",)),
    )(x, yt)
    return out[0, 0]
